# baseline (speedup 1.0000x reference)
import functools

import jax
import jax.numpy as jnp
from jax import lax
from jax.experimental import pallas as pl
from jax.experimental.pallas import tpu as pltpu
from jax.experimental.pallas import tpu_sc as plsc

N = 10000
DF = 128
K = 60

_IMASK = 0x3FFF
_IMAXI = 2**31 - 1


_NW = 32


def _gather_rows_sc(table, idx3):
    g = idx3.shape[1]
    out_rows = _NW * g * 128
    mesh = plsc.VectorSubcoreMesh(core_axis_name="c", subcore_axis_name="s")

    @functools.partial(
        pl.kernel, mesh=mesh,
        compiler_params=pltpu.CompilerParams(use_tc_tiling_on_sc=False),
        out_type=jax.ShapeDtypeStruct((out_rows, 64), jnp.float32),
        scratch_types=(
            [pltpu.VMEM((g, 128), jnp.int32)]
            + [pltpu.VMEM((128, 64), jnp.float32) for _ in range(4)]
            + [pltpu.SemaphoreType.DMA for _ in range(4)]),
    )
    def k(table_hbm, idx_hbm, out_hbm, idxv, r0, r1, r2, r3, s0, s1, s2, s3):
        wid = lax.axis_index("s") * 2 + lax.axis_index("c")
        pltpu.sync_copy(idx_hbm.at[wid], idxv)
        rbufs = (r0, r1, r2, r3)
        sems = (s0, s1, s2, s3)
        base = wid * (g * 128)

        def body(i, _):
            g0 = i * 4
            cps = [pltpu.async_copy(table_hbm.at[idxv.at[g0 + b]],
                                    rbufs[b], sems[b]) for b in range(4)]
            for b in range(4):
                cps[b].wait()
                pltpu.sync_copy(
                    rbufs[b], out_hbm.at[pl.ds(base + (g0 + b) * 128, 128)])
            return 0

        lax.fori_loop(0, g // 4, body, 0)

    return k(table, idx3)


def _sc_gather(table, idx):
    e = idx.shape[0]
    g = -(-e // (_NW * 128))
    g += (-g) % 4
    tot = _NW * g * 128
    idxp = jnp.concatenate(
        [idx.astype(jnp.int32), jnp.zeros((tot - e,), jnp.int32)]
    ).reshape(_NW, g, 128)
    return _gather_rows_sc(table, idxp)[:e]


def _matmul_body(x_ref, w_ref, o_ref):
    o_ref[...] = jnp.dot(x_ref[...], w_ref[...],
                         preferred_element_type=jnp.float32)


def _node_linear(x, w):
    n, fin = x.shape
    fout = w.shape[1]
    return pl.pallas_call(
        _matmul_body,
        out_shape=jax.ShapeDtypeStruct((n, fout), jnp.float32),
    )(x, w)


_KNN_R = 200
_NPAD = 10240
_TOPP = 8


def _knn_body(xb_ref, xt_ref, x2_ref, o_ref, key_ref):
    i = pl.program_id(0)
    xb = xb_ref[...]
    d = (x2_ref[...]
         - 2.0 * jnp.dot(xb, xt_ref[...], preferred_element_type=jnp.float32)
         + jnp.sum(xb * xb, axis=1, keepdims=True))
    bits = jax.lax.bitcast_convert_type(d, jnp.int32)
    skey = bits ^ ((bits >> 31) & jnp.int32(0x7FFFFFFF))
    jcol = jax.lax.broadcasted_iota(jnp.int32, d.shape, 1)
    rowg = jax.lax.broadcasted_iota(jnp.int32, d.shape, 0) + i * _KNN_R
    key = (skey & jnp.int32(~_IMASK)) | jcol
    key_ref[...] = jnp.where((jcol == rowg) | (jcol >= N),
                             jnp.int32(_IMAXI), key)

    m = [jnp.full((_KNN_R, 128), _IMAXI, jnp.int32) for _ in range(_TOPP)]
    for c in range(_NPAD // 128):
        a = key_ref[:, c * 128:(c + 1) * 128]
        for t in range(_TOPP):
            lo = jnp.minimum(m[t], a)
            a = jnp.maximum(m[t], a)
            m[t] = lo

    cand = jnp.concatenate(m, axis=1)
    lane = jax.lax.broadcasted_iota(jnp.int32, (_KNN_R, 64), 1)
    acc0 = jnp.zeros((_KNN_R, 64), jnp.int32)

    def body(t, carry):
        acc, k = carry
        kmin = jnp.min(k, axis=1, keepdims=True)
        acc = jnp.where(lane == t, kmin & _IMASK, acc)
        return acc, jnp.where(k == kmin, jnp.int32(_IMAXI), k)

    acc, _ = jax.lax.fori_loop(0, K, body, (acc0, cand))
    o_ref[...] = acc


def _knn(x):
    n, f = x.shape
    xt = jnp.concatenate(
        [x.T, jnp.zeros((f, _NPAD - n), jnp.float32)], axis=1)
    x2 = jnp.concatenate(
        [jnp.sum(x * x, axis=1), jnp.zeros((_NPAD - n,), jnp.float32)]
    ).reshape(1, _NPAD)
    grid = n // _KNN_R
    nbr = pl.pallas_call(
        _knn_body,
        grid=(grid,),
        in_specs=[
            pl.BlockSpec((_KNN_R, f), lambda i: (i, 0)),
            pl.BlockSpec((f, _NPAD), lambda i: (0, 0)),
            pl.BlockSpec((1, _NPAD), lambda i: (0, 0)),
        ],
        out_specs=pl.BlockSpec((_KNN_R, 64), lambda i: (i, 0)),
        out_shape=jax.ShapeDtypeStruct((n, 64), jnp.int32),
        scratch_shapes=[
            pltpu.VMEM((_KNN_R, _NPAD), jnp.int32),
        ],
    )(x, xt, x2)
    return nbr[:, :K]


_EC_R = 400


def _ec2_body(g_ref, h_ref, b1_ref, w2_ref, b2_ref, o_ref):
    h3 = jnp.broadcast_to(h_ref[...][:, None, :], (_EC_R, K, 64))
    m = jax.nn.relu(g_ref[...].reshape(_EC_R, K, 64) - h3
                    + b1_ref[...].reshape(1, 1, 64))
    y = jax.nn.relu(
        jnp.dot(m.reshape(_EC_R * K, 64), w2_ref[...],
                preferred_element_type=jnp.float32) + b2_ref[...])
    o_ref[...] = jnp.max(y.reshape(_EC_R, K, 64), axis=1)


def _edgeconv_knn2(gath, h, b1, w2, b2):
    grid = N // _EC_R
    return pl.pallas_call(
        _ec2_body,
        grid=(grid,),
        in_specs=[
            pl.BlockSpec((_EC_R * K, 64), lambda i: (i, 0)),
            pl.BlockSpec((_EC_R, 64), lambda i: (i, 0)),
            pl.BlockSpec((1, 64), lambda i: (0, 0)),
            pl.BlockSpec((64, 64), lambda i: (0, 0)),
            pl.BlockSpec((1, 64), lambda i: (0, 0)),
        ],
        out_specs=pl.BlockSpec((_EC_R, 64), lambda i: (i, 0)),
        out_shape=jax.ShapeDtypeStruct((N, 64), jnp.float32),
    )(gath, h, b1.reshape(1, 64), w2, b2.reshape(1, 64))


def _ec1l_body(g_ref, h_ref, b_ref, o_ref):
    h3 = jnp.broadcast_to(h_ref[...][:, None, :], (_EC_R, K, 64))
    m = jax.nn.relu(g_ref[...].reshape(_EC_R, K, 64) - h3
                    + b_ref[...].reshape(1, 1, 64))
    o_ref[...] = jnp.max(m, axis=1)


def _edgeconv_knn1(gath, h, b):
    grid = N // _EC_R
    return pl.pallas_call(
        _ec1l_body,
        grid=(grid,),
        in_specs=[
            pl.BlockSpec((_EC_R * K, 64), lambda i: (i, 0)),
            pl.BlockSpec((_EC_R, 64), lambda i: (i, 0)),
            pl.BlockSpec((1, 64), lambda i: (0, 0)),
        ],
        out_specs=pl.BlockSpec((_EC_R, 64), lambda i: (i, 0)),
        out_shape=jax.ShapeDtypeStruct((N, 64), jnp.float32),
    )(gath, h, b.reshape(1, 64))


_E1B = 4000


def _ec1msg_body(gs_ref, gd_ref, b1_ref, w2_ref, b2_ref, o_ref):
    m = jax.nn.relu(gs_ref[...] - gd_ref[...] + b1_ref[...])
    o_ref[...] = jax.nn.relu(
        jnp.dot(m, w2_ref[...], preferred_element_type=jnp.float32)
        + b2_ref[...])


def _ec1_messages(gs, gd, b1, w2, b2):
    e = gs.shape[0]
    grid = e // _E1B
    return pl.pallas_call(
        _ec1msg_body,
        grid=(grid,),
        in_specs=[
            pl.BlockSpec((_E1B, 64), lambda i: (i, 0)),
            pl.BlockSpec((_E1B, 64), lambda i: (i, 0)),
            pl.BlockSpec((1, 64), lambda i: (0, 0)),
            pl.BlockSpec((64, 64), lambda i: (0, 0)),
            pl.BlockSpec((1, 64), lambda i: (0, 0)),
        ],
        out_specs=pl.BlockSpec((_E1B, 64), lambda i: (i, 0)),
        out_shape=jax.ShapeDtypeStruct((e, 64), jnp.float32),
    )(gs, gd, b1.reshape(1, 64), w2, b2.reshape(1, 64))


_ROWS = 400


def _fc_head_body(s1_ref, lab_ref, w1, b1, w2, b2, w3, b3, w4, b4, w5, b5,
                  out_ref, loss_ref):
    i = pl.program_id(0)
    s1 = s1_ref[...]
    a = jax.nn.relu(jnp.dot(s1, w1[...], preferred_element_type=jnp.float32)
                    + b1[...])
    w2a = w2[0:192, :]
    w2b = w2[192:, :]
    h = jax.nn.relu(jnp.dot(s1, w2a, preferred_element_type=jnp.float32)
                    + jnp.dot(a, w2b, preferred_element_type=jnp.float32)
                    + b2[...])
    h = jax.nn.relu(jnp.dot(h, w3[...], preferred_element_type=jnp.float32)
                    + b3[...])
    h = jax.nn.relu(jnp.dot(h, w4[...], preferred_element_type=jnp.float32)
                    + b4[...])
    logit = jnp.dot(h, w5[...], preferred_element_type=jnp.float32) + b5[...]
    bprob = jax.nn.sigmoid(logit)
    out_ref[...] = bprob
    p = jnp.clip(bprob[:, 0], 1e-7, 1.0 - 1e-7)
    lab = lab_ref[0, 0, :]
    ll = lab * jnp.log(p) + (1.0 - lab) * jnp.log(1.0 - p)
    part = jnp.sum(ll)

    @pl.when(i == 0)
    def _():
        loss_ref[...] = jnp.zeros_like(loss_ref)

    loss_ref[...] += part.reshape(1, 1)


def _fc_head(stack1, labels, params):
    grid = N // _ROWS
    fc = [params["fc1"], params["fc2"], params["fc3"], params["fc4"],
          params["fc5"]]
    wb = []
    for p in fc:
        wb.append(p["W"])
        wb.append(p["b"].reshape(1, -1))
    lab2 = labels.reshape(grid, 1, _ROWS)
    out, losssum = pl.pallas_call(
        _fc_head_body,
        grid=(grid,),
        in_specs=[
            pl.BlockSpec((_ROWS, 192), lambda i: (i, 0)),
            pl.BlockSpec((1, 1, _ROWS), lambda i: (i, 0, 0)),
        ] + [pl.BlockSpec(w.shape, lambda i: (0, 0)) for w in wb],
        out_specs=[
            pl.BlockSpec((_ROWS, 1), lambda i: (i, 0)),
            pl.BlockSpec((1, 1), lambda i: (0, 0)),
        ],
        out_shape=[
            jax.ShapeDtypeStruct((N, 1), jnp.float32),
            jax.ShapeDtypeStruct((1, 1), jnp.float32),
        ],
    )(stack1, lab2, *wb)
    loss = -losssum[0, 0] / N
    return loss, out


def kernel(x, edge_index, labels, params):
    p1, p2, p3 = params["ec1"], params["ec2"], params["ec3"]

    h1 = _node_linear(x, p1["lin1"]["W"])
    src, dst = edge_index[0], edge_index[1]
    e1 = src.shape[0]
    gsd = _sc_gather(h1, jnp.concatenate([src, dst]))
    gs = gsd[:e1]
    gd = gsd[e1:]
    msg = _ec1_messages(gs, gd, p1["lin1"]["b"], p1["lin2"]["W"],
                        p1["lin2"]["b"])
    agg = jax.ops.segment_max(msg, dst, num_segments=N)
    xo = jnp.where(jnp.isfinite(agg), agg, 0.0)

    nbr2 = _knn(xo)
    h2 = _node_linear(xo, p2["lin1"]["W"])
    g2 = _sc_gather(h2, nbr2.reshape(-1))
    y = _edgeconv_knn2(g2, h2, p2["lin1"]["b"], p2["lin2"]["W"],
                       p2["lin2"]["b"])

    nbr3 = _knn(y)
    h3 = _node_linear(y, p3["lin"]["W"])
    g3 = _sc_gather(h3, nbr3.reshape(-1))
    z = _edgeconv_knn1(g3, h3, p3["lin"]["b"])

    stack1 = jnp.concatenate([xo, y, z], axis=-1)
    return _fc_head(stack1, labels, params)


# speedup vs baseline: 1.1179x; 1.1179x over previous
import functools

import jax
import jax.numpy as jnp
from jax import lax
from jax.experimental import pallas as pl
from jax.experimental.pallas import tpu as pltpu
from jax.experimental.pallas import tpu_sc as plsc

N = 10000
DF = 128
K = 60

_IMASK = 0x3FFF
_IMAXI = 2**31 - 1


_NW = 32


def _gather_rows_sc(table, idx3):
    g = idx3.shape[1]
    out_rows = _NW * g * 128
    mesh = plsc.VectorSubcoreMesh(core_axis_name="c", subcore_axis_name="s")

    @functools.partial(
        pl.kernel, mesh=mesh,
        compiler_params=pltpu.CompilerParams(use_tc_tiling_on_sc=False),
        out_type=jax.ShapeDtypeStruct((out_rows, 64), jnp.float32),
        scratch_types=(
            [pltpu.VMEM((g, 128), jnp.int32)]
            + [pltpu.VMEM((128, 64), jnp.float32) for _ in range(8)]
            + [pltpu.SemaphoreType.DMA for _ in range(8)]),
    )
    def k(table_hbm, idx_hbm, out_hbm, idxv,
          r0, r1, r2, r3, r4, r5, r6, r7,
          s0, s1, s2, s3, s4, s5, s6, s7):
        wid = lax.axis_index("s") * 2 + lax.axis_index("c")
        pltpu.sync_copy(idx_hbm.at[wid], idxv)
        rbufs = (r0, r1, r2, r3, r4, r5, r6, r7)
        sems = (s0, s1, s2, s3, s4, s5, s6, s7)
        base = wid * (g * 128)
        ngroups = g // 4

        def fire(grp, s):
            for b in range(4):
                pltpu.async_copy(table_hbm.at[idxv.at[grp * 4 + b]],
                                 rbufs[s * 4 + b], sems[s * 4 + b])

        def drain(grp, s):
            for b in range(4):
                pltpu.make_async_copy(
                    table_hbm.at[idxv.at[grp * 4 + b]],
                    rbufs[s * 4 + b], sems[s * 4 + b]).wait()
                pltpu.sync_copy(
                    rbufs[s * 4 + b],
                    out_hbm.at[pl.ds(base + (grp * 4 + b) * 128, 128)])

        fire(0, 0)

        def body(i, _):
            s = lax.rem(i, 2)

            @pl.when(i + 1 < ngroups)
            def _():
                @pl.when(s == 0)
                def _():
                    fire(i + 1, 1)

                @pl.when(s == 1)
                def _():
                    fire(i + 1, 0)

            @pl.when(s == 0)
            def _():
                drain(i, 0)

            @pl.when(s == 1)
            def _():
                drain(i, 1)

            return 0

        lax.fori_loop(0, ngroups, body, 0)

    return k(table, idx3)


def _sc_gather(table, idx):
    e = idx.shape[0]
    g = -(-e // (_NW * 128))
    g += (-g) % 4
    tot = _NW * g * 128
    idxp = jnp.concatenate(
        [idx.astype(jnp.int32), jnp.zeros((tot - e,), jnp.int32)]
    ).reshape(_NW, g, 128)
    return _gather_rows_sc(table, idxp)[:e]


def _matmul_body(x_ref, w_ref, o_ref):
    o_ref[...] = jnp.dot(x_ref[...], w_ref[...],
                         preferred_element_type=jnp.float32)


def _node_linear(x, w):
    n, fin = x.shape
    fout = w.shape[1]
    return pl.pallas_call(
        _matmul_body,
        out_shape=jax.ShapeDtypeStruct((n, fout), jnp.float32),
    )(x, w)


_KNN_R = 200
_NPAD = 10240
_TOPP = 8


def _knn_body(xb_ref, xt_ref, x2_ref, o_ref, key_ref, cand_ref, acc_ref):
    i = pl.program_id(0)
    xb = xb_ref[...]
    d = (x2_ref[...]
         - 2.0 * jnp.dot(xb, xt_ref[...], preferred_element_type=jnp.float32)
         + jnp.sum(xb * xb, axis=1, keepdims=True))
    bits = jax.lax.bitcast_convert_type(d, jnp.int32)
    skey = bits ^ ((bits >> 31) & jnp.int32(0x7FFFFFFF))
    jcol = jax.lax.broadcasted_iota(jnp.int32, d.shape, 1)
    rowg = jax.lax.broadcasted_iota(jnp.int32, d.shape, 0) + i * _KNN_R
    key = (skey & jnp.int32(~_IMASK)) | jcol
    key_ref[...] = jnp.where((jcol == rowg) | (jcol >= N),
                             jnp.int32(_IMAXI), key)

    m = [jnp.full((_KNN_R, 128), _IMAXI, jnp.int32) for _ in range(_TOPP)]
    for c in range(_NPAD // 128):
        a = key_ref[:, c * 128:(c + 1) * 128]
        for t in range(_TOPP):
            lo = jnp.minimum(m[t], a)
            a = jnp.maximum(m[t], a)
            m[t] = lo

    cand_ref[...] = jnp.concatenate(m, axis=1)
    lane = jax.lax.broadcasted_iota(jnp.int32, acc_ref.shape, 1)

    def body(t, _):
        k = cand_ref[...]
        kmin = jnp.min(k, axis=1, keepdims=True)
        acc_ref[...] = jnp.where(lane == t, kmin & _IMASK, acc_ref[...])
        cand_ref[...] = jnp.where(k == kmin, jnp.int32(_IMAXI), k)
        return 0

    jax.lax.fori_loop(0, K, body, 0)
    o_ref[...] = acc_ref[...]


def _knn(x):
    n, f = x.shape
    xt = jnp.concatenate(
        [x.T, jnp.zeros((f, _NPAD - n), jnp.float32)], axis=1)
    x2 = jnp.concatenate(
        [jnp.sum(x * x, axis=1), jnp.zeros((_NPAD - n,), jnp.float32)]
    ).reshape(1, _NPAD)
    grid = n // _KNN_R
    nbr = pl.pallas_call(
        _knn_body,
        grid=(grid,),
        in_specs=[
            pl.BlockSpec((_KNN_R, f), lambda i: (i, 0)),
            pl.BlockSpec((f, _NPAD), lambda i: (0, 0)),
            pl.BlockSpec((1, _NPAD), lambda i: (0, 0)),
        ],
        out_specs=pl.BlockSpec((_KNN_R, 64), lambda i: (i, 0)),
        out_shape=jax.ShapeDtypeStruct((n, 64), jnp.int32),
        scratch_shapes=[
            pltpu.VMEM((_KNN_R, _NPAD), jnp.int32),
            pltpu.VMEM((_KNN_R, 128 * _TOPP), jnp.int32),
            pltpu.VMEM((_KNN_R, 64), jnp.int32),
        ],
    )(x, xt, x2)
    return nbr[:, :K]


_EC_R = 400


def _ec2_body(g_ref, h_ref, b1_ref, w2_ref, b2_ref, o_ref):
    h3 = jnp.broadcast_to(h_ref[...][:, None, :], (_EC_R, K, 64))
    m = jax.nn.relu(g_ref[...].reshape(_EC_R, K, 64) - h3
                    + b1_ref[...].reshape(1, 1, 64))
    y = jax.nn.relu(
        jnp.dot(m.reshape(_EC_R * K, 64), w2_ref[...],
                preferred_element_type=jnp.float32) + b2_ref[...])
    o_ref[...] = jnp.max(y.reshape(_EC_R, K, 64), axis=1)


def _edgeconv_knn2(gath, h, b1, w2, b2):
    grid = N // _EC_R
    return pl.pallas_call(
        _ec2_body,
        grid=(grid,),
        in_specs=[
            pl.BlockSpec((_EC_R * K, 64), lambda i: (i, 0)),
            pl.BlockSpec((_EC_R, 64), lambda i: (i, 0)),
            pl.BlockSpec((1, 64), lambda i: (0, 0)),
            pl.BlockSpec((64, 64), lambda i: (0, 0)),
            pl.BlockSpec((1, 64), lambda i: (0, 0)),
        ],
        out_specs=pl.BlockSpec((_EC_R, 64), lambda i: (i, 0)),
        out_shape=jax.ShapeDtypeStruct((N, 64), jnp.float32),
    )(gath, h, b1.reshape(1, 64), w2, b2.reshape(1, 64))


def _ec1l_body(g_ref, h_ref, b_ref, o_ref):
    h3 = jnp.broadcast_to(h_ref[...][:, None, :], (_EC_R, K, 64))
    m = jax.nn.relu(g_ref[...].reshape(_EC_R, K, 64) - h3
                    + b_ref[...].reshape(1, 1, 64))
    o_ref[...] = jnp.max(m, axis=1)


def _edgeconv_knn1(gath, h, b):
    grid = N // _EC_R
    return pl.pallas_call(
        _ec1l_body,
        grid=(grid,),
        in_specs=[
            pl.BlockSpec((_EC_R * K, 64), lambda i: (i, 0)),
            pl.BlockSpec((_EC_R, 64), lambda i: (i, 0)),
            pl.BlockSpec((1, 64), lambda i: (0, 0)),
        ],
        out_specs=pl.BlockSpec((_EC_R, 64), lambda i: (i, 0)),
        out_shape=jax.ShapeDtypeStruct((N, 64), jnp.float32),
    )(gath, h, b.reshape(1, 64))


_E1B = 4000


def _ec1msg_body(gs_ref, gd_ref, b1_ref, w2_ref, b2_ref, o_ref):
    m = jax.nn.relu(gs_ref[...] - gd_ref[...] + b1_ref[...])
    o_ref[...] = jax.nn.relu(
        jnp.dot(m, w2_ref[...], preferred_element_type=jnp.float32)
        + b2_ref[...])


def _ec1_messages(gs, gd, b1, w2, b2):
    e = gs.shape[0]
    grid = e // _E1B
    return pl.pallas_call(
        _ec1msg_body,
        grid=(grid,),
        in_specs=[
            pl.BlockSpec((_E1B, 64), lambda i: (i, 0)),
            pl.BlockSpec((_E1B, 64), lambda i: (i, 0)),
            pl.BlockSpec((1, 64), lambda i: (0, 0)),
            pl.BlockSpec((64, 64), lambda i: (0, 0)),
            pl.BlockSpec((1, 64), lambda i: (0, 0)),
        ],
        out_specs=pl.BlockSpec((_E1B, 64), lambda i: (i, 0)),
        out_shape=jax.ShapeDtypeStruct((e, 64), jnp.float32),
    )(gs, gd, b1.reshape(1, 64), w2, b2.reshape(1, 64))


_ROWS = 400


def _fc_head_body(s1_ref, lab_ref, w1, b1, w2, b2, w3, b3, w4, b4, w5, b5,
                  out_ref, loss_ref):
    i = pl.program_id(0)
    s1 = s1_ref[...]
    a = jax.nn.relu(jnp.dot(s1, w1[...], preferred_element_type=jnp.float32)
                    + b1[...])
    w2a = w2[0:192, :]
    w2b = w2[192:, :]
    h = jax.nn.relu(jnp.dot(s1, w2a, preferred_element_type=jnp.float32)
                    + jnp.dot(a, w2b, preferred_element_type=jnp.float32)
                    + b2[...])
    h = jax.nn.relu(jnp.dot(h, w3[...], preferred_element_type=jnp.float32)
                    + b3[...])
    h = jax.nn.relu(jnp.dot(h, w4[...], preferred_element_type=jnp.float32)
                    + b4[...])
    logit = jnp.dot(h, w5[...], preferred_element_type=jnp.float32) + b5[...]
    bprob = jax.nn.sigmoid(logit)
    out_ref[...] = bprob
    p = jnp.clip(bprob[:, 0], 1e-7, 1.0 - 1e-7)
    lab = lab_ref[0, 0, :]
    ll = lab * jnp.log(p) + (1.0 - lab) * jnp.log(1.0 - p)
    part = jnp.sum(ll)

    @pl.when(i == 0)
    def _():
        loss_ref[...] = jnp.zeros_like(loss_ref)

    loss_ref[...] += part.reshape(1, 1)


def _fc_head(stack1, labels, params):
    grid = N // _ROWS
    fc = [params["fc1"], params["fc2"], params["fc3"], params["fc4"],
          params["fc5"]]
    wb = []
    for p in fc:
        wb.append(p["W"])
        wb.append(p["b"].reshape(1, -1))
    lab2 = labels.reshape(grid, 1, _ROWS)
    out, losssum = pl.pallas_call(
        _fc_head_body,
        grid=(grid,),
        in_specs=[
            pl.BlockSpec((_ROWS, 192), lambda i: (i, 0)),
            pl.BlockSpec((1, 1, _ROWS), lambda i: (i, 0, 0)),
        ] + [pl.BlockSpec(w.shape, lambda i: (0, 0)) for w in wb],
        out_specs=[
            pl.BlockSpec((_ROWS, 1), lambda i: (i, 0)),
            pl.BlockSpec((1, 1), lambda i: (0, 0)),
        ],
        out_shape=[
            jax.ShapeDtypeStruct((N, 1), jnp.float32),
            jax.ShapeDtypeStruct((1, 1), jnp.float32),
        ],
    )(stack1, lab2, *wb)
    loss = -losssum[0, 0] / N
    return loss, out


def kernel(x, edge_index, labels, params):
    p1, p2, p3 = params["ec1"], params["ec2"], params["ec3"]

    h1 = _node_linear(x, p1["lin1"]["W"])
    src, dst = edge_index[0], edge_index[1]
    gs = _sc_gather(h1, src)
    gd = _sc_gather(h1, dst)
    msg = _ec1_messages(gs, gd, p1["lin1"]["b"], p1["lin2"]["W"],
                        p1["lin2"]["b"])
    agg = jax.ops.segment_max(msg, dst, num_segments=N)
    xo = jnp.where(jnp.isfinite(agg), agg, 0.0)

    nbr2 = _knn(xo)
    h2 = _node_linear(xo, p2["lin1"]["W"])
    g2 = _sc_gather(h2, nbr2.reshape(-1))
    y = _edgeconv_knn2(g2, h2, p2["lin1"]["b"], p2["lin2"]["W"],
                       p2["lin2"]["b"])

    nbr3 = _knn(y)
    h3 = _node_linear(y, p3["lin"]["W"])
    g3 = _sc_gather(h3, nbr3.reshape(-1))
    z = _edgeconv_knn1(g3, h3, p3["lin"]["b"])

    stack1 = jnp.concatenate([xo, y, z], axis=-1)
    return _fc_head(stack1, labels, params)


# speedup vs baseline: 1.1496x; 1.0284x over previous
import functools

import jax
import jax.numpy as jnp
from jax import lax
from jax.experimental import pallas as pl
from jax.experimental.pallas import tpu as pltpu
from jax.experimental.pallas import tpu_sc as plsc

N = 10000
DF = 128
K = 60

_IMASK = 0x3FFF
_IMAXI = 2**31 - 1


_NW = 32


def _gather_rows_sc(table, idx3):
    g = idx3.shape[1]
    out_rows = _NW * g * 128
    mesh = plsc.VectorSubcoreMesh(core_axis_name="c", subcore_axis_name="s")

    @functools.partial(
        pl.kernel, mesh=mesh,
        compiler_params=pltpu.CompilerParams(use_tc_tiling_on_sc=False),
        out_type=jax.ShapeDtypeStruct((out_rows, 64), jnp.float32),
        scratch_types=(
            [pltpu.VMEM((g, 128), jnp.int32)]
            + [pltpu.VMEM((128, 64), jnp.float32) for _ in range(8)]
            + [pltpu.SemaphoreType.DMA for _ in range(8)]),
    )
    def k(table_hbm, idx_hbm, out_hbm, idxv,
          r0, r1, r2, r3, r4, r5, r6, r7,
          s0, s1, s2, s3, s4, s5, s6, s7):
        wid = lax.axis_index("s") * 2 + lax.axis_index("c")
        pltpu.sync_copy(idx_hbm.at[wid], idxv)
        rbufs = (r0, r1, r2, r3, r4, r5, r6, r7)
        sems = (s0, s1, s2, s3, s4, s5, s6, s7)
        base = wid * (g * 128)
        ngroups = g // 4

        def fire(grp, s):
            for b in range(4):
                pltpu.async_copy(table_hbm.at[idxv.at[grp * 4 + b]],
                                 rbufs[s * 4 + b], sems[s * 4 + b])

        def drain(grp, s):
            for b in range(4):
                pltpu.make_async_copy(
                    table_hbm.at[idxv.at[grp * 4 + b]],
                    rbufs[s * 4 + b], sems[s * 4 + b]).wait()
                pltpu.sync_copy(
                    rbufs[s * 4 + b],
                    out_hbm.at[pl.ds(base + (grp * 4 + b) * 128, 128)])

        fire(0, 0)

        def body(i, _):
            s = lax.rem(i, 2)

            @pl.when(i + 1 < ngroups)
            def _():
                @pl.when(s == 0)
                def _():
                    fire(i + 1, 1)

                @pl.when(s == 1)
                def _():
                    fire(i + 1, 0)

            @pl.when(s == 0)
            def _():
                drain(i, 0)

            @pl.when(s == 1)
            def _():
                drain(i, 1)

            return 0

        lax.fori_loop(0, ngroups, body, 0)

    return k(table, idx3)


def _sc_gather(table, idx):
    e = idx.shape[0]
    g = -(-e // (_NW * 128))
    g += (-g) % 4
    tot = _NW * g * 128
    idxp = jnp.concatenate(
        [idx.astype(jnp.int32), jnp.zeros((tot - e,), jnp.int32)]
    ).reshape(_NW, g, 128)
    return _gather_rows_sc(table, idxp)[:e]


def _matmul_body(x_ref, w_ref, o_ref):
    o_ref[...] = jnp.dot(x_ref[...], w_ref[...],
                         preferred_element_type=jnp.float32)


def _node_linear(x, w):
    n, fin = x.shape
    fout = w.shape[1]
    return pl.pallas_call(
        _matmul_body,
        out_shape=jax.ShapeDtypeStruct((n, fout), jnp.float32),
    )(x, w)


_KNN_R = 200
_NPAD = 10240
_TOPP = 5


def _knn_body(xb_ref, xt_ref, x2_ref, o_ref, key_ref, cand_ref, acc_ref):
    i = pl.program_id(0)
    xb = xb_ref[...]
    d = (x2_ref[...]
         - 2.0 * jnp.dot(xb, xt_ref[...], preferred_element_type=jnp.float32)
         + jnp.sum(xb * xb, axis=1, keepdims=True))
    bits = jax.lax.bitcast_convert_type(d, jnp.int32)
    skey = bits ^ ((bits >> 31) & jnp.int32(0x7FFFFFFF))
    jcol = jax.lax.broadcasted_iota(jnp.int32, d.shape, 1)
    rowg = jax.lax.broadcasted_iota(jnp.int32, d.shape, 0) + i * _KNN_R
    key = (skey & jnp.int32(~_IMASK)) | jcol
    key_ref[...] = jnp.where((jcol == rowg) | (jcol >= N),
                             jnp.int32(_IMAXI), key)

    banks = [[jnp.full((_KNN_R, 128), _IMAXI, jnp.int32)
              for _ in range(_TOPP)] for _ in range(2)]
    for c in range(_NPAD // 128):
        a = key_ref[:, c * 128:(c + 1) * 128]
        m = banks[c % 2]
        for t in range(_TOPP):
            lo = jnp.minimum(m[t], a)
            if t < _TOPP - 1:
                a = jnp.maximum(m[t], a)
            m[t] = lo

    cand_ref[...] = jnp.concatenate(banks[0] + banks[1], axis=1)
    lane = jax.lax.broadcasted_iota(jnp.int32, acc_ref.shape, 1)

    def body(t, _):
        k = cand_ref[...]
        kmin = jnp.min(k, axis=1, keepdims=True)
        acc_ref[...] = jnp.where(lane == t, kmin & _IMASK, acc_ref[...])
        cand_ref[...] = jnp.where(k == kmin, jnp.int32(_IMAXI), k)
        return 0

    jax.lax.fori_loop(0, K, body, 0)
    o_ref[...] = acc_ref[...]


def _knn(x):
    n, f = x.shape
    xt = jnp.concatenate(
        [x.T, jnp.zeros((f, _NPAD - n), jnp.float32)], axis=1)
    x2 = jnp.concatenate(
        [jnp.sum(x * x, axis=1), jnp.zeros((_NPAD - n,), jnp.float32)]
    ).reshape(1, _NPAD)
    grid = n // _KNN_R
    nbr = pl.pallas_call(
        _knn_body,
        grid=(grid,),
        in_specs=[
            pl.BlockSpec((_KNN_R, f), lambda i: (i, 0)),
            pl.BlockSpec((f, _NPAD), lambda i: (0, 0)),
            pl.BlockSpec((1, _NPAD), lambda i: (0, 0)),
        ],
        out_specs=pl.BlockSpec((_KNN_R, 64), lambda i: (i, 0)),
        out_shape=jax.ShapeDtypeStruct((n, 64), jnp.int32),
        scratch_shapes=[
            pltpu.VMEM((_KNN_R, _NPAD), jnp.int32),
            pltpu.VMEM((_KNN_R, 256 * _TOPP), jnp.int32),
            pltpu.VMEM((_KNN_R, 64), jnp.int32),
        ],
    )(x, xt, x2)
    return nbr[:, :K]


_EC_R = 400


def _ec2_body(g_ref, h_ref, b1_ref, w2_ref, b2_ref, o_ref):
    h3 = jnp.broadcast_to(h_ref[...][:, None, :], (_EC_R, K, 64))
    m = jax.nn.relu(g_ref[...].reshape(_EC_R, K, 64) - h3
                    + b1_ref[...].reshape(1, 1, 64))
    y = jax.nn.relu(
        jnp.dot(m.reshape(_EC_R * K, 64), w2_ref[...],
                preferred_element_type=jnp.float32) + b2_ref[...])
    o_ref[...] = jnp.max(y.reshape(_EC_R, K, 64), axis=1)


def _edgeconv_knn2(gath, h, b1, w2, b2):
    grid = N // _EC_R
    return pl.pallas_call(
        _ec2_body,
        grid=(grid,),
        in_specs=[
            pl.BlockSpec((_EC_R * K, 64), lambda i: (i, 0)),
            pl.BlockSpec((_EC_R, 64), lambda i: (i, 0)),
            pl.BlockSpec((1, 64), lambda i: (0, 0)),
            pl.BlockSpec((64, 64), lambda i: (0, 0)),
            pl.BlockSpec((1, 64), lambda i: (0, 0)),
        ],
        out_specs=pl.BlockSpec((_EC_R, 64), lambda i: (i, 0)),
        out_shape=jax.ShapeDtypeStruct((N, 64), jnp.float32),
    )(gath, h, b1.reshape(1, 64), w2, b2.reshape(1, 64))


def _ec1l_body(g_ref, h_ref, b_ref, o_ref):
    h3 = jnp.broadcast_to(h_ref[...][:, None, :], (_EC_R, K, 64))
    m = jax.nn.relu(g_ref[...].reshape(_EC_R, K, 64) - h3
                    + b_ref[...].reshape(1, 1, 64))
    o_ref[...] = jnp.max(m, axis=1)


def _edgeconv_knn1(gath, h, b):
    grid = N // _EC_R
    return pl.pallas_call(
        _ec1l_body,
        grid=(grid,),
        in_specs=[
            pl.BlockSpec((_EC_R * K, 64), lambda i: (i, 0)),
            pl.BlockSpec((_EC_R, 64), lambda i: (i, 0)),
            pl.BlockSpec((1, 64), lambda i: (0, 0)),
        ],
        out_specs=pl.BlockSpec((_EC_R, 64), lambda i: (i, 0)),
        out_shape=jax.ShapeDtypeStruct((N, 64), jnp.float32),
    )(gath, h, b.reshape(1, 64))


_E1B = 4000


def _ec1msg_body(gs_ref, gd_ref, b1_ref, w2_ref, b2_ref, o_ref):
    m = jax.nn.relu(gs_ref[...] - gd_ref[...] + b1_ref[...])
    o_ref[...] = jax.nn.relu(
        jnp.dot(m, w2_ref[...], preferred_element_type=jnp.float32)
        + b2_ref[...])


def _ec1_messages(gs, gd, b1, w2, b2):
    e = gs.shape[0]
    grid = e // _E1B
    return pl.pallas_call(
        _ec1msg_body,
        grid=(grid,),
        in_specs=[
            pl.BlockSpec((_E1B, 64), lambda i: (i, 0)),
            pl.BlockSpec((_E1B, 64), lambda i: (i, 0)),
            pl.BlockSpec((1, 64), lambda i: (0, 0)),
            pl.BlockSpec((64, 64), lambda i: (0, 0)),
            pl.BlockSpec((1, 64), lambda i: (0, 0)),
        ],
        out_specs=pl.BlockSpec((_E1B, 64), lambda i: (i, 0)),
        out_shape=jax.ShapeDtypeStruct((e, 64), jnp.float32),
    )(gs, gd, b1.reshape(1, 64), w2, b2.reshape(1, 64))


_ROWS = 400


def _fc_head_body(s1_ref, lab_ref, w1, b1, w2, b2, w3, b3, w4, b4, w5, b5,
                  out_ref, loss_ref):
    i = pl.program_id(0)
    s1 = s1_ref[...]
    a = jax.nn.relu(jnp.dot(s1, w1[...], preferred_element_type=jnp.float32)
                    + b1[...])
    w2a = w2[0:192, :]
    w2b = w2[192:, :]
    h = jax.nn.relu(jnp.dot(s1, w2a, preferred_element_type=jnp.float32)
                    + jnp.dot(a, w2b, preferred_element_type=jnp.float32)
                    + b2[...])
    h = jax.nn.relu(jnp.dot(h, w3[...], preferred_element_type=jnp.float32)
                    + b3[...])
    h = jax.nn.relu(jnp.dot(h, w4[...], preferred_element_type=jnp.float32)
                    + b4[...])
    logit = jnp.dot(h, w5[...], preferred_element_type=jnp.float32) + b5[...]
    bprob = jax.nn.sigmoid(logit)
    out_ref[...] = bprob
    p = jnp.clip(bprob[:, 0], 1e-7, 1.0 - 1e-7)
    lab = lab_ref[0, 0, :]
    ll = lab * jnp.log(p) + (1.0 - lab) * jnp.log(1.0 - p)
    part = jnp.sum(ll)

    @pl.when(i == 0)
    def _():
        loss_ref[...] = jnp.zeros_like(loss_ref)

    loss_ref[...] += part.reshape(1, 1)


def _fc_head(stack1, labels, params):
    grid = N // _ROWS
    fc = [params["fc1"], params["fc2"], params["fc3"], params["fc4"],
          params["fc5"]]
    wb = []
    for p in fc:
        wb.append(p["W"])
        wb.append(p["b"].reshape(1, -1))
    lab2 = labels.reshape(grid, 1, _ROWS)
    out, losssum = pl.pallas_call(
        _fc_head_body,
        grid=(grid,),
        in_specs=[
            pl.BlockSpec((_ROWS, 192), lambda i: (i, 0)),
            pl.BlockSpec((1, 1, _ROWS), lambda i: (i, 0, 0)),
        ] + [pl.BlockSpec(w.shape, lambda i: (0, 0)) for w in wb],
        out_specs=[
            pl.BlockSpec((_ROWS, 1), lambda i: (i, 0)),
            pl.BlockSpec((1, 1), lambda i: (0, 0)),
        ],
        out_shape=[
            jax.ShapeDtypeStruct((N, 1), jnp.float32),
            jax.ShapeDtypeStruct((1, 1), jnp.float32),
        ],
    )(stack1, lab2, *wb)
    loss = -losssum[0, 0] / N
    return loss, out


def kernel(x, edge_index, labels, params):
    p1, p2, p3 = params["ec1"], params["ec2"], params["ec3"]

    h1 = _node_linear(x, p1["lin1"]["W"])
    src, dst = edge_index[0], edge_index[1]
    gs = _sc_gather(h1, src)
    gd = _sc_gather(h1, dst)
    msg = _ec1_messages(gs, gd, p1["lin1"]["b"], p1["lin2"]["W"],
                        p1["lin2"]["b"])
    agg = jax.ops.segment_max(msg, dst, num_segments=N)
    xo = jnp.where(jnp.isfinite(agg), agg, 0.0)

    nbr2 = _knn(xo)
    h2 = _node_linear(xo, p2["lin1"]["W"])
    g2 = _sc_gather(h2, nbr2.reshape(-1))
    y = _edgeconv_knn2(g2, h2, p2["lin1"]["b"], p2["lin2"]["W"],
                       p2["lin2"]["b"])

    nbr3 = _knn(y)
    h3 = _node_linear(y, p3["lin"]["W"])
    g3 = _sc_gather(h3, nbr3.reshape(-1))
    z = _edgeconv_knn1(g3, h3, p3["lin"]["b"])

    stack1 = jnp.concatenate([xo, y, z], axis=-1)
    return _fc_head(stack1, labels, params)


# speedup vs baseline: 1.2761x; 1.1101x over previous
import functools

import jax
import jax.numpy as jnp
from jax import lax
from jax.experimental import pallas as pl
from jax.experimental.pallas import tpu as pltpu
from jax.experimental.pallas import tpu_sc as plsc

N = 10000
DF = 128
K = 60

_IMASK = 0x3FFF
_IMAXI = 2**31 - 1


_NW = 32


def _gather_rows_sc(table, idx3):
    g = idx3.shape[1]
    out_rows = _NW * g * 128
    mesh = plsc.VectorSubcoreMesh(core_axis_name="c", subcore_axis_name="s")

    @functools.partial(
        pl.kernel, mesh=mesh,
        compiler_params=pltpu.CompilerParams(use_tc_tiling_on_sc=False),
        out_type=jax.ShapeDtypeStruct((out_rows, 64), jnp.float32),
        scratch_types=(
            [pltpu.VMEM((g, 128), jnp.int32)]
            + [pltpu.VMEM((128, 64), jnp.float32) for _ in range(8)]
            + [pltpu.SemaphoreType.DMA for _ in range(8)]),
    )
    def k(table_hbm, idx_hbm, out_hbm, idxv,
          r0, r1, r2, r3, r4, r5, r6, r7,
          s0, s1, s2, s3, s4, s5, s6, s7):
        wid = lax.axis_index("s") * 2 + lax.axis_index("c")
        pltpu.sync_copy(idx_hbm.at[wid], idxv)
        rbufs = (r0, r1, r2, r3, r4, r5, r6, r7)
        sems = (s0, s1, s2, s3, s4, s5, s6, s7)
        base = wid * (g * 128)
        ngroups = g // 4

        def fire(grp, s):
            for b in range(4):
                pltpu.async_copy(table_hbm.at[idxv.at[grp * 4 + b]],
                                 rbufs[s * 4 + b], sems[s * 4 + b])

        def drain(grp, s):
            for b in range(4):
                pltpu.make_async_copy(
                    table_hbm.at[idxv.at[grp * 4 + b]],
                    rbufs[s * 4 + b], sems[s * 4 + b]).wait()
                pltpu.sync_copy(
                    rbufs[s * 4 + b],
                    out_hbm.at[pl.ds(base + (grp * 4 + b) * 128, 128)])

        fire(0, 0)

        def body(i, _):
            s = lax.rem(i, 2)

            @pl.when(i + 1 < ngroups)
            def _():
                @pl.when(s == 0)
                def _():
                    fire(i + 1, 1)

                @pl.when(s == 1)
                def _():
                    fire(i + 1, 0)

            @pl.when(s == 0)
            def _():
                drain(i, 0)

            @pl.when(s == 1)
            def _():
                drain(i, 1)

            return 0

        lax.fori_loop(0, ngroups, body, 0)

    return k(table, idx3)


def _sc_gather(table, idx):
    e = idx.shape[0]
    g = -(-e // (_NW * 128))
    g += (-g) % 4
    tot = _NW * g * 128
    idxp = jnp.concatenate(
        [idx.astype(jnp.int32), jnp.zeros((tot - e,), jnp.int32)]
    ).reshape(_NW, g, 128)
    return _gather_rows_sc(table, idxp)


def _matmul_body(x_ref, w_ref, o_ref):
    o_ref[...] = jnp.dot(x_ref[...], w_ref[...],
                         preferred_element_type=jnp.float32)


def _node_linear(x, w):
    n, fin = x.shape
    fout = w.shape[1]
    return pl.pallas_call(
        _matmul_body,
        out_shape=jax.ShapeDtypeStruct((n, fout), jnp.float32),
    )(x, w)


_KNN_R = 200
_NPAD = 10240
_TOPP = 5


def _knn_body(xb_ref, xt_ref, x2_ref, o_ref, key_ref, cand_ref, acc_ref):
    i = pl.program_id(0)
    xb = xb_ref[...]
    d = (x2_ref[...]
         - 2.0 * jnp.dot(xb, xt_ref[...], preferred_element_type=jnp.float32)
         + jnp.sum(xb * xb, axis=1, keepdims=True))
    bits = jax.lax.bitcast_convert_type(d, jnp.int32)
    skey = bits ^ ((bits >> 31) & jnp.int32(0x7FFFFFFF))
    jcol = jax.lax.broadcasted_iota(jnp.int32, d.shape, 1)
    rowg = jax.lax.broadcasted_iota(jnp.int32, d.shape, 0) + i * _KNN_R
    key = (skey & jnp.int32(~_IMASK)) | jcol
    key_ref[...] = jnp.where((jcol == rowg) | (jcol >= N),
                             jnp.int32(_IMAXI), key)

    banks = [[jnp.full((_KNN_R, 128), _IMAXI, jnp.int32)
              for _ in range(_TOPP)] for _ in range(2)]
    for c in range(_NPAD // 128):
        a = key_ref[:, c * 128:(c + 1) * 128]
        m = banks[c % 2]
        for t in range(_TOPP):
            lo = jnp.minimum(m[t], a)
            if t < _TOPP - 1:
                a = jnp.maximum(m[t], a)
            m[t] = lo

    cand_ref[...] = jnp.concatenate(banks[0] + banks[1], axis=1)
    lane = jax.lax.broadcasted_iota(jnp.int32, acc_ref.shape, 1)

    def body(t, _):
        k = cand_ref[...]
        kmin = jnp.min(k, axis=1, keepdims=True)
        acc_ref[...] = jnp.where(lane == t, kmin & _IMASK, acc_ref[...])
        cand_ref[...] = jnp.where(k == kmin, jnp.int32(_IMAXI), k)
        return 0

    jax.lax.fori_loop(0, K, body, 0)
    o_ref[...] = acc_ref[...]


def _knn(x):
    n, f = x.shape
    xt = jnp.concatenate(
        [x.T, jnp.zeros((f, _NPAD - n), jnp.float32)], axis=1)
    x2 = jnp.concatenate(
        [jnp.sum(x * x, axis=1), jnp.zeros((_NPAD - n,), jnp.float32)]
    ).reshape(1, _NPAD)
    grid = n // _KNN_R
    nbr = pl.pallas_call(
        _knn_body,
        grid=(grid,),
        in_specs=[
            pl.BlockSpec((_KNN_R, f), lambda i: (i, 0)),
            pl.BlockSpec((f, _NPAD), lambda i: (0, 0)),
            pl.BlockSpec((1, _NPAD), lambda i: (0, 0)),
        ],
        out_specs=pl.BlockSpec((_KNN_R, 64), lambda i: (i, 0)),
        out_shape=jax.ShapeDtypeStruct((n, 64), jnp.int32),
        scratch_shapes=[
            pltpu.VMEM((_KNN_R, _NPAD), jnp.int32),
            pltpu.VMEM((_KNN_R, 256 * _TOPP), jnp.int32),
            pltpu.VMEM((_KNN_R, 64), jnp.int32),
        ],
    )(x, xt, x2)
    return nbr[:, :K]


_EC_R = 400


def _ec2_body(g_ref, h_ref, b1_ref, w2_ref, b2_ref, o_ref):
    h3 = jnp.broadcast_to(h_ref[...][:, None, :], (_EC_R, K, 64))
    m = jax.nn.relu(g_ref[...].reshape(_EC_R, K, 64) - h3
                    + b1_ref[...].reshape(1, 1, 64))
    y = jax.nn.relu(
        jnp.dot(m.reshape(_EC_R * K, 64), w2_ref[...],
                preferred_element_type=jnp.float32) + b2_ref[...])
    o_ref[...] = jnp.max(y.reshape(_EC_R, K, 64), axis=1)


def _edgeconv_knn2(gath, h, b1, w2, b2):
    grid = N // _EC_R
    return pl.pallas_call(
        _ec2_body,
        grid=(grid,),
        in_specs=[
            pl.BlockSpec((_EC_R * K, 64), lambda i: (i, 0)),
            pl.BlockSpec((_EC_R, 64), lambda i: (i, 0)),
            pl.BlockSpec((1, 64), lambda i: (0, 0)),
            pl.BlockSpec((64, 64), lambda i: (0, 0)),
            pl.BlockSpec((1, 64), lambda i: (0, 0)),
        ],
        out_specs=pl.BlockSpec((_EC_R, 64), lambda i: (i, 0)),
        out_shape=jax.ShapeDtypeStruct((N, 64), jnp.float32),
    )(gath, h, b1.reshape(1, 64), w2, b2.reshape(1, 64))


def _ec1l_body(g_ref, h_ref, b_ref, o_ref):
    h3 = jnp.broadcast_to(h_ref[...][:, None, :], (_EC_R, K, 64))
    m = jax.nn.relu(g_ref[...].reshape(_EC_R, K, 64) - h3
                    + b_ref[...].reshape(1, 1, 64))
    o_ref[...] = jnp.max(m, axis=1)


def _edgeconv_knn1(gath, h, b):
    grid = N // _EC_R
    return pl.pallas_call(
        _ec1l_body,
        grid=(grid,),
        in_specs=[
            pl.BlockSpec((_EC_R * K, 64), lambda i: (i, 0)),
            pl.BlockSpec((_EC_R, 64), lambda i: (i, 0)),
            pl.BlockSpec((1, 64), lambda i: (0, 0)),
        ],
        out_specs=pl.BlockSpec((_EC_R, 64), lambda i: (i, 0)),
        out_shape=jax.ShapeDtypeStruct((N, 64), jnp.float32),
    )(gath, h, b.reshape(1, 64))


_E1B = 4000


def _ec1msg_body(gs_ref, gd_ref, b1_ref, w2_ref, b2_ref, o_ref):
    m = jax.nn.relu(gs_ref[...] - gd_ref[...] + b1_ref[...])
    o_ref[...] = jax.nn.relu(
        jnp.dot(m, w2_ref[...], preferred_element_type=jnp.float32)
        + b2_ref[...])


def _ec1_messages(gs, gd, b1, w2, b2, e):
    grid = e // _E1B
    return pl.pallas_call(
        _ec1msg_body,
        grid=(grid,),
        in_specs=[
            pl.BlockSpec((_E1B, 64), lambda i: (i, 0)),
            pl.BlockSpec((_E1B, 64), lambda i: (i, 0)),
            pl.BlockSpec((1, 64), lambda i: (0, 0)),
            pl.BlockSpec((64, 64), lambda i: (0, 0)),
            pl.BlockSpec((1, 64), lambda i: (0, 0)),
        ],
        out_specs=pl.BlockSpec((_E1B, 64), lambda i: (i, 0)),
        out_shape=jax.ShapeDtypeStruct((e, 64), jnp.float32),
    )(gs, gd, b1.reshape(1, 64), w2, b2.reshape(1, 64))


_ROWS = 400


def _fc_head_body(s1_ref, lab_ref, w1, b1, w2, b2, w3, b3, w4, b4, w5, b5,
                  out_ref, loss_ref):
    i = pl.program_id(0)
    s1 = s1_ref[...]
    a = jax.nn.relu(jnp.dot(s1, w1[...], preferred_element_type=jnp.float32)
                    + b1[...])
    w2a = w2[0:192, :]
    w2b = w2[192:, :]
    h = jax.nn.relu(jnp.dot(s1, w2a, preferred_element_type=jnp.float32)
                    + jnp.dot(a, w2b, preferred_element_type=jnp.float32)
                    + b2[...])
    h = jax.nn.relu(jnp.dot(h, w3[...], preferred_element_type=jnp.float32)
                    + b3[...])
    h = jax.nn.relu(jnp.dot(h, w4[...], preferred_element_type=jnp.float32)
                    + b4[...])
    logit = jnp.dot(h, w5[...], preferred_element_type=jnp.float32) + b5[...]
    bprob = jax.nn.sigmoid(logit)
    out_ref[...] = bprob
    p = jnp.clip(bprob[:, 0], 1e-7, 1.0 - 1e-7)
    lab = lab_ref[0, 0, :]
    ll = lab * jnp.log(p) + (1.0 - lab) * jnp.log(1.0 - p)
    part = jnp.sum(ll)

    @pl.when(i == 0)
    def _():
        loss_ref[...] = jnp.zeros_like(loss_ref)

    loss_ref[...] += part.reshape(1, 1)


def _fc_head(stack1, labels, params):
    grid = N // _ROWS
    fc = [params["fc1"], params["fc2"], params["fc3"], params["fc4"],
          params["fc5"]]
    wb = []
    for p in fc:
        wb.append(p["W"])
        wb.append(p["b"].reshape(1, -1))
    lab2 = labels.reshape(grid, 1, _ROWS)
    out, losssum = pl.pallas_call(
        _fc_head_body,
        grid=(grid,),
        in_specs=[
            pl.BlockSpec((_ROWS, 192), lambda i: (i, 0)),
            pl.BlockSpec((1, 1, _ROWS), lambda i: (i, 0, 0)),
        ] + [pl.BlockSpec(w.shape, lambda i: (0, 0)) for w in wb],
        out_specs=[
            pl.BlockSpec((_ROWS, 1), lambda i: (i, 0)),
            pl.BlockSpec((1, 1), lambda i: (0, 0)),
        ],
        out_shape=[
            jax.ShapeDtypeStruct((N, 1), jnp.float32),
            jax.ShapeDtypeStruct((1, 1), jnp.float32),
        ],
    )(stack1, lab2, *wb)
    loss = -losssum[0, 0] / N
    return loss, out


def kernel(x, edge_index, labels, params):
    p1, p2, p3 = params["ec1"], params["ec2"], params["ec3"]

    h1 = _node_linear(x, p1["lin1"]["W"])
    src, dst = edge_index[0], edge_index[1]
    gs = _sc_gather(h1, src)
    gd = _sc_gather(h1, dst)
    msg = _ec1_messages(gs, gd, p1["lin1"]["b"], p1["lin2"]["W"],
                        p1["lin2"]["b"], src.shape[0])
    agg = jax.ops.segment_max(msg, dst, num_segments=N)
    xo = jnp.where(jnp.isfinite(agg), agg, 0.0)

    nbr2 = _knn(xo)
    h2 = _node_linear(xo, p2["lin1"]["W"])
    g2 = _sc_gather(h2, nbr2.reshape(-1))
    y = _edgeconv_knn2(g2, h2, p2["lin1"]["b"], p2["lin2"]["W"],
                       p2["lin2"]["b"])

    nbr3 = _knn(y)
    h3 = _node_linear(y, p3["lin"]["W"])
    g3 = _sc_gather(h3, nbr3.reshape(-1))
    z = _edgeconv_knn1(g3, h3, p3["lin"]["b"])

    stack1 = jnp.concatenate([xo, y, z], axis=-1)
    return _fc_head(stack1, labels, params)
